# baseline (device time: 868106 ns/iter reference)
import jax
import jax.numpy as jnp
from jax import lax
from jax.experimental import pallas as pl
from jax.experimental.pallas import tpu as pltpu

T = 1024
D = 2048
V_HALF = 16384
V = 2 * V_HALF
TILE = 512
N_TILES = V_HALF // TILE
K = 9
SLANES = 128


def _gemm_headsend(x, W):

    def body(x_ref, w_ref, logits_ref, c_ref, nbr_raw_ref,
             m_ref, s_ref, head, stat_buf, stat_recv,
             head_send_sems, head_recv_sems, stat_send_sem, stat_recv_sem):
        j = pl.program_id(0)
        my_x = lax.axis_index("x")
        my_y = lax.axis_index("y")
        nbr = (my_x, 1 - my_y)

        logits = jnp.dot(x_ref[...], w_ref[...],
                         preferred_element_type=jnp.float32)
        logits_ref[...] = logits
        m_t = jnp.max(logits, axis=1, keepdims=True)
        s_t = jnp.sum(jnp.exp(logits - m_t), axis=1, keepdims=True)

        @pl.when(j == 0)
        def _():
            m_ref[...] = jnp.broadcast_to(m_t, (T, SLANES))
            s_ref[...] = jnp.broadcast_to(s_t, (T, SLANES))

        @pl.when(j > 0)
        def _():
            m_old = m_ref[:, :1]
            s_old = s_ref[:, :1]
            m_new = jnp.maximum(m_old, m_t)
            s_new = (s_old * jnp.exp(m_old - m_new)
                     + s_t * jnp.exp(m_t - m_new))
            m_ref[...] = jnp.broadcast_to(m_new, (T, SLANES))
            s_ref[...] = jnp.broadcast_to(s_new, (T, SLANES))

        def head_rdma(i):
            return pltpu.make_async_remote_copy(
                src_ref=head.at[i],
                dst_ref=nbr_raw_ref.at[:, pl.ds(i * TILE, TILE)],
                send_sem=head_send_sems.at[i],
                recv_sem=head_recv_sems.at[i],
                device_id=nbr,
                device_id_type=pl.DeviceIdType.MESH,
            )

        for i in range(K):
            @pl.when(j == i)
            def _(i=i):
                head[i, :, :] = logits
                head_rdma(i).start()

        @pl.when(j == N_TILES - 1)
        def _():
            for i in range(K):
                head_rdma(i).wait()
            stat_buf[0, :, :] = m_ref[...]
            stat_buf[1, :, :] = s_ref[...]
            rs = pltpu.make_async_remote_copy(
                src_ref=stat_buf,
                dst_ref=stat_recv,
                send_sem=stat_send_sem,
                recv_sem=stat_recv_sem,
                device_id=nbr,
                device_id_type=pl.DeviceIdType.MESH,
            )
            rs.start()
            rs.wait()
            m_l = m_ref[:, :1]
            s_l = s_ref[:, :1]
            m_o = stat_recv[0, :, :1]
            s_o = stat_recv[1, :, :1]
            m_g = jnp.maximum(m_l, m_o)
            z_g = s_l * jnp.exp(m_l - m_g) + s_o * jnp.exp(m_o - m_g)
            c_ref[...] = jnp.broadcast_to(m_g + jnp.log(z_g), (T, SLANES))

    return pl.pallas_call(
        body,
        grid=(N_TILES,),
        in_specs=[
            pl.BlockSpec((T, D), lambda j: (0, 0)),
            pl.BlockSpec((D, TILE), lambda j: (0, j)),
        ],
        out_specs=[
            pl.BlockSpec((T, TILE), lambda j: (0, j)),
            pl.BlockSpec((T, SLANES), lambda j: (0, 0)),
            pl.BlockSpec(memory_space=pl.ANY),
        ],
        out_shape=[
            jax.ShapeDtypeStruct((T, V_HALF), jnp.float32),
            jax.ShapeDtypeStruct((T, SLANES), jnp.float32),
            jax.ShapeDtypeStruct((T, K * TILE), jnp.float32),
        ],
        scratch_shapes=[
            pltpu.VMEM((T, SLANES), jnp.float32),
            pltpu.VMEM((T, SLANES), jnp.float32),
            pltpu.VMEM((K, T, TILE), jnp.float32),
            pltpu.VMEM((2, T, SLANES), jnp.float32),
            pltpu.VMEM((2, T, SLANES), jnp.float32),
            pltpu.SemaphoreType.DMA((K,)),
            pltpu.SemaphoreType.DMA((K,)),
            pltpu.SemaphoreType.DMA,
            pltpu.SemaphoreType.DMA,
        ],
        compiler_params=pltpu.CompilerParams(
            has_side_effects=True, vmem_limit_bytes=100 * 1024 * 1024),
    )(x, W)


def _normalize_exchange(logits, c, nbr_raw):

    N_SEND = N_TILES - K

    def body(logits_ref, c_ref, nbr_raw_ref, out_ref,
             snd, raw_t, cp_sems, send_sems, recv_sems, raw_ld_sem,
             raw_st_sem):
        j = pl.program_id(0)
        jt = (j + K) % N_TILES
        slot = lax.rem(j, 2)
        my_x = lax.axis_index("x")
        my_y = lax.axis_index("y")
        nbr = (my_x, 1 - my_y)
        my_col = my_y * V_HALF + jt * TILE
        nbr_col = (1 - my_y) * V_HALF + jt * TILE

        def local_cp(sl, col):
            return pltpu.make_async_copy(
                snd.at[sl], out_ref.at[:, pl.ds(col, TILE)], cp_sems.at[sl])

        def send_rdma(sl, col, tile_idx):
            return pltpu.make_async_remote_copy(
                src_ref=snd.at[sl],
                dst_ref=out_ref.at[:, pl.ds(col, TILE)],
                send_sem=send_sems.at[sl],
                recv_sem=recv_sems.at[tile_idx],
                device_id=nbr,
                device_id_type=pl.DeviceIdType.MESH,
            )

        @pl.when(j >= 2)
        def _():
            jt_prev = (j - 2 + K) % N_TILES
            col_prev = my_y * V_HALF + jt_prev * TILE
            local_cp(slot, col_prev).wait()

            @pl.when(j - 2 < N_SEND)
            def _():
                send_rdma(slot, col_prev, jt_prev).wait_send()

        snd[slot, :, :] = jnp.exp(logits_ref[...] - c_ref[:, :1])
        local_cp(slot, my_col).start()

        @pl.when(j < N_SEND)
        def _():
            send_rdma(slot, my_col, jt).start()

        @pl.when(j >= N_SEND)
        def _():
            ld = pltpu.make_async_copy(
                nbr_raw_ref.at[:, pl.ds(jt * TILE, TILE)], raw_t, raw_ld_sem)
            ld.start()
            ld.wait()
            raw_t[...] = jnp.exp(raw_t[...] - c_ref[:, :1])
            st = pltpu.make_async_copy(
                raw_t, out_ref.at[:, pl.ds(nbr_col, TILE)], raw_st_sem)
            st.start()
            st.wait()

        @pl.when(j == N_TILES - 1)
        def _():
            for dj in (N_TILES - 2, N_TILES - 1):
                sl = dj % 2
                jtp = (dj + K) % N_TILES
                local_cp(sl, my_y * V_HALF + jtp * TILE).wait()
            for i in range(K, N_TILES):
                r = pltpu.make_async_remote_copy(
                    src_ref=snd.at[0],
                    dst_ref=out_ref.at[
                        :, pl.ds((1 - my_y) * V_HALF + i * TILE, TILE)],
                    send_sem=send_sems.at[0],
                    recv_sem=recv_sems.at[i],
                    device_id=nbr,
                    device_id_type=pl.DeviceIdType.MESH,
                )
                r.wait_recv()

    return pl.pallas_call(
        body,
        grid=(N_TILES,),
        in_specs=[
            pl.BlockSpec((T, TILE), lambda j: (0, (j + K) % N_TILES)),
            pl.BlockSpec((T, SLANES), lambda j: (0, 0)),
            pl.BlockSpec(memory_space=pl.ANY),
        ],
        out_specs=pl.BlockSpec(memory_space=pltpu.MemorySpace.HBM),
        out_shape=jax.ShapeDtypeStruct((T, V), jnp.float32),
        scratch_shapes=[
            pltpu.VMEM((2, T, TILE), jnp.float32),
            pltpu.VMEM((T, TILE), jnp.float32),
            pltpu.SemaphoreType.DMA((2,)),
            pltpu.SemaphoreType.DMA((2,)),
            pltpu.SemaphoreType.DMA((N_TILES,)),
            pltpu.SemaphoreType.DMA,
            pltpu.SemaphoreType.DMA,
        ],
        compiler_params=pltpu.CompilerParams(has_side_effects=True),
    )(logits, c, nbr_raw)


def kernel(x, W):
    logits, c, nbr_raw = _gemm_headsend(x, W)
    return _normalize_exchange(logits, c, nbr_raw)


# device time: 592447 ns/iter; 1.4653x vs baseline; 1.4653x over previous
import jax
import jax.numpy as jnp
from jax import lax
from jax.experimental import pallas as pl
from jax.experimental.pallas import tpu as pltpu

T = 1024
D = 2048
V_HALF = 16384
V = 2 * V_HALF
TILE = 512
N_TILES = V_HALF // TILE
PAIRS = N_TILES // 2
K_H = 10
D_N = PAIRS - K_H
SLANES = 128

_FWD_STEP = [10, 13, 16, 18, 20, 22, 25, 27, 29, 31][:K_H]


def _gemm_headsend(x, W):

    def body(x_ref, w_ref, logits_ref, c_ref, nbr_raw_ref, nbr_raw_x_ref,
             m_ref, s_ref, head, stat_buf, stat_recv,
             hd_send_sems, hd_recv_sems, fwdr_send_sems, fwdr_recv_sems,
             stat_send_sem, stat_recv_sem):
        j = pl.program_id(0)
        my_x = lax.axis_index("x")
        my_y = lax.axis_index("y")
        ynbr = (my_x, 1 - my_y)
        xnbr = (1 - my_x, my_y)

        logits = jnp.dot(x_ref[...], w_ref[...],
                         preferred_element_type=jnp.float32)
        logits_ref[...] = logits
        m_t = jnp.max(logits, axis=1, keepdims=True)
        s_t = jnp.sum(jnp.exp(logits - m_t), axis=1, keepdims=True)

        @pl.when(j == 0)
        def _():
            m_ref[...] = jnp.broadcast_to(m_t, (T, SLANES))
            s_ref[...] = jnp.broadcast_to(s_t, (T, SLANES))

        @pl.when(j > 0)
        def _():
            m_old = m_ref[:, :1]
            s_old = s_ref[:, :1]
            m_new = jnp.maximum(m_old, m_t)
            s_new = (s_old * jnp.exp(m_old - m_new)
                     + s_t * jnp.exp(m_t - m_new))
            m_ref[...] = jnp.broadcast_to(m_new, (T, SLANES))
            s_ref[...] = jnp.broadcast_to(s_new, (T, SLANES))

        def head_rdma(k):
            return pltpu.make_async_remote_copy(
                src_ref=head.at[k],
                dst_ref=nbr_raw_ref.at[:, pl.ds(k * TILE, TILE)],
                send_sem=hd_send_sems.at[k],
                recv_sem=hd_recv_sems.at[k],
                device_id=ynbr,
                device_id_type=pl.DeviceIdType.MESH,
            )

        def fwd_rdma(k):
            return pltpu.make_async_remote_copy(
                src_ref=nbr_raw_ref.at[:, pl.ds(k * TILE, TILE)],
                dst_ref=nbr_raw_x_ref.at[:, pl.ds(k * TILE, TILE)],
                send_sem=fwdr_send_sems.at[k],
                recv_sem=fwdr_recv_sems.at[k],
                device_id=xnbr,
                device_id_type=pl.DeviceIdType.MESH,
            )

        for k in range(K_H):
            @pl.when(j == 2 * k + my_x)
            def _(k=k):
                head[k, :, :] = logits
                head_rdma(k).start()

        for k in range(K_H):
            @pl.when(j == _FWD_STEP[k])
            def _(k=k):
                head_rdma(k).wait_recv()
                fwd_rdma(k).start()

        @pl.when(j == N_TILES - 1)
        def _():
            for k in range(K_H):
                head_rdma(k).wait_send()
                fwd_rdma(k).wait_send()
                fwd_rdma(k).wait_recv()
            stat_buf[0, :, :] = m_ref[...]
            stat_buf[1, :, :] = s_ref[...]
            rs = pltpu.make_async_remote_copy(
                src_ref=stat_buf,
                dst_ref=stat_recv,
                send_sem=stat_send_sem,
                recv_sem=stat_recv_sem,
                device_id=ynbr,
                device_id_type=pl.DeviceIdType.MESH,
            )
            rs.start()
            rs.wait()
            m_l = m_ref[:, :1]
            s_l = s_ref[:, :1]
            m_o = stat_recv[0, :, :1]
            s_o = stat_recv[1, :, :1]
            m_g = jnp.maximum(m_l, m_o)
            z_g = s_l * jnp.exp(m_l - m_g) + s_o * jnp.exp(m_o - m_g)
            c_ref[...] = jnp.broadcast_to(m_g + jnp.log(z_g), (T, SLANES))

    return pl.pallas_call(
        body,
        grid=(N_TILES,),
        in_specs=[
            pl.BlockSpec((T, D), lambda j: (0, 0)),
            pl.BlockSpec((D, TILE), lambda j: (0, j)),
        ],
        out_specs=[
            pl.BlockSpec((T, TILE), lambda j: (0, j)),
            pl.BlockSpec((T, SLANES), lambda j: (0, 0)),
            pl.BlockSpec(memory_space=pl.ANY),
            pl.BlockSpec(memory_space=pl.ANY),
        ],
        out_shape=[
            jax.ShapeDtypeStruct((T, V_HALF), jnp.float32),
            jax.ShapeDtypeStruct((T, SLANES), jnp.float32),
            jax.ShapeDtypeStruct((T, K_H * TILE), jnp.float32),
            jax.ShapeDtypeStruct((T, K_H * TILE), jnp.float32),
        ],
        scratch_shapes=[
            pltpu.VMEM((T, SLANES), jnp.float32),
            pltpu.VMEM((T, SLANES), jnp.float32),
            pltpu.VMEM((K_H, T, TILE), jnp.float32),
            pltpu.VMEM((2, T, SLANES), jnp.float32),
            pltpu.VMEM((2, T, SLANES), jnp.float32),
            pltpu.SemaphoreType.DMA((K_H,)),
            pltpu.SemaphoreType.DMA((K_H,)),
            pltpu.SemaphoreType.DMA((K_H,)),
            pltpu.SemaphoreType.DMA((K_H,)),
            pltpu.SemaphoreType.DMA,
            pltpu.SemaphoreType.DMA,
        ],
        compiler_params=pltpu.CompilerParams(
            has_side_effects=True, vmem_limit_bytes=100 * 1024 * 1024),
    )(x, W)


def _normalize_exchange(logits, c, nbr_raw, nbr_raw_x):

    def body(logits_ref, c_ref, nbr_raw_ref, nbr_raw_x_ref, out_ref,
             snd, rawy_t, rawx_t, cp_sems, snd_send_sems, d_recv_sems,
             fwd_send_sems, fwd_recv_sems, rawy_ld, rawy_st, rawx_ld,
             rawx_st):
        j = pl.program_id(0)
        kp = lax.rem(j + K_H, PAIRS)
        slot = lax.rem(j, 2)
        my_x = lax.axis_index("x")
        my_y = lax.axis_index("y")
        ynbr = (my_x, 1 - my_y)
        xnbr = (1 - my_x, my_y)
        my_col = my_y * V_HALF + kp * 2 * TILE
        dcol = my_col + my_x * TILE

        def pair_cp(sl, col):
            return pltpu.make_async_copy(
                snd.at[sl], out_ref.at[:, pl.ds(col, 2 * TILE)],
                cp_sems.at[sl])

        def direct_send(sl, col, pair_idx):
            return pltpu.make_async_remote_copy(
                src_ref=snd.at[sl, :, pl.ds(my_x * TILE, TILE)],
                dst_ref=out_ref.at[:, pl.ds(col, TILE)],
                send_sem=snd_send_sems.at[sl],
                recv_sem=d_recv_sems.at[pair_idx],
                device_id=ynbr,
                device_id_type=pl.DeviceIdType.MESH,
            )

        def fwd_rdma(pair_idx):
            fcol = (1 - my_y) * V_HALF + pair_idx * 2 * TILE + my_x * TILE
            return pltpu.make_async_remote_copy(
                src_ref=out_ref.at[:, pl.ds(fcol, TILE)],
                dst_ref=out_ref.at[:, pl.ds(fcol, TILE)],
                send_sem=fwd_send_sems.at[pair_idx],
                recv_sem=fwd_recv_sems.at[pair_idx],
                device_id=xnbr,
                device_id_type=pl.DeviceIdType.MESH,
            )

        @pl.when(j >= 2)
        def _():
            kp2 = lax.rem(j - 2 + K_H, PAIRS)
            col2 = my_y * V_HALF + kp2 * 2 * TILE
            pair_cp(slot, col2).wait()

            @pl.when(j - 2 < D_N)
            def _():
                direct_send(slot, col2 + my_x * TILE, kp2).wait_send()

        snd[slot, :, :] = jnp.exp(logits_ref[...] - c_ref[:, :1])
        pair_cp(slot, my_col).start()

        @pl.when(j < D_N)
        def _():
            direct_send(slot, dcol, kp).start()

        @pl.when((j >= 2) & (j < 2 + D_N))
        def _():
            kf = j - 2 + K_H
            fcol = (1 - my_y) * V_HALF + kf * 2 * TILE + my_x * TILE
            arr = pltpu.make_async_remote_copy(
                src_ref=snd.at[0, :, pl.ds(0, TILE)],
                dst_ref=out_ref.at[:, pl.ds(fcol, TILE)],
                send_sem=snd_send_sems.at[0],
                recv_sem=d_recv_sems.at[kf],
                device_id=ynbr,
                device_id_type=pl.DeviceIdType.MESH,
            )
            arr.wait_recv()
            fwd_rdma(kf).start()

        @pl.when(j >= D_N)
        def _():
            ycol = (1 - my_y) * V_HALF + (2 * kp + my_x) * TILE
            ld = pltpu.make_async_copy(
                nbr_raw_ref.at[:, pl.ds(kp * TILE, TILE)], rawy_t, rawy_ld)
            ld.start()
            ld.wait()
            rawy_t[...] = jnp.exp(rawy_t[...] - c_ref[:, :1])
            st = pltpu.make_async_copy(
                rawy_t, out_ref.at[:, pl.ds(ycol, TILE)], rawy_st)
            st.start()
            xcol = (1 - my_y) * V_HALF + (2 * kp + 1 - my_x) * TILE
            ld2 = pltpu.make_async_copy(
                nbr_raw_x_ref.at[:, pl.ds(kp * TILE, TILE)], rawx_t, rawx_ld)
            ld2.start()
            ld2.wait()
            rawx_t[...] = jnp.exp(rawx_t[...] - c_ref[:, :1])
            st2 = pltpu.make_async_copy(
                rawx_t, out_ref.at[:, pl.ds(xcol, TILE)], rawx_st)
            st2.start()
            st.wait()
            st2.wait()

        @pl.when(j == PAIRS - 1)
        def _():
            for dj in (PAIRS - 2, PAIRS - 1):
                sl = dj % 2
                kpd = (dj + K_H) % PAIRS
                pair_cp(sl, my_y * V_HALF + kpd * 2 * TILE).wait()
            for k in range(K_H, PAIRS):
                fwd_rdma(k).wait_send()
                fcol_in = ((1 - my_y) * V_HALF + k * 2 * TILE
                           + (1 - my_x) * TILE)
                arr = pltpu.make_async_remote_copy(
                    src_ref=snd.at[0, :, pl.ds(0, TILE)],
                    dst_ref=out_ref.at[:, pl.ds(fcol_in, TILE)],
                    send_sem=snd_send_sems.at[0],
                    recv_sem=fwd_recv_sems.at[k],
                    device_id=xnbr,
                    device_id_type=pl.DeviceIdType.MESH,
                )
                arr.wait_recv()

    return pl.pallas_call(
        body,
        grid=(PAIRS,),
        in_specs=[
            pl.BlockSpec((T, 2 * TILE), lambda j: (0, (j + K_H) % PAIRS)),
            pl.BlockSpec((T, SLANES), lambda j: (0, 0)),
            pl.BlockSpec(memory_space=pl.ANY),
            pl.BlockSpec(memory_space=pl.ANY),
        ],
        out_specs=pl.BlockSpec(memory_space=pl.ANY),
        out_shape=jax.ShapeDtypeStruct((T, V), jnp.float32),
        scratch_shapes=[
            pltpu.VMEM((2, T, 2 * TILE), jnp.float32),
            pltpu.VMEM((T, TILE), jnp.float32),
            pltpu.VMEM((T, TILE), jnp.float32),
            pltpu.SemaphoreType.DMA((2,)),
            pltpu.SemaphoreType.DMA((2,)),
            pltpu.SemaphoreType.DMA((PAIRS,)),
            pltpu.SemaphoreType.DMA((PAIRS,)),
            pltpu.SemaphoreType.DMA((PAIRS,)),
            pltpu.SemaphoreType.DMA,
            pltpu.SemaphoreType.DMA,
            pltpu.SemaphoreType.DMA,
            pltpu.SemaphoreType.DMA,
        ],
        compiler_params=pltpu.CompilerParams(
            has_side_effects=True, vmem_limit_bytes=64 * 1024 * 1024),
    )(logits, c, nbr_raw, nbr_raw_x)


def kernel(x, W):
    logits, c, nbr_raw, nbr_raw_x = _gemm_headsend(x, W)
    return _normalize_exchange(logits, c, nbr_raw, nbr_raw_x)


# device time: 580403 ns/iter; 1.4957x vs baseline; 1.0208x over previous
import jax
import jax.numpy as jnp
from jax import lax
from jax.experimental import pallas as pl
from jax.experimental.pallas import tpu as pltpu

T = 1024
D = 2048
V_HALF = 16384
V = 2 * V_HALF
TILE = 512
N_TILES = V_HALF // TILE
PAIRS = N_TILES // 2
K_H = 10
D_N = PAIRS - K_H
SLANES = 128

_FWD_STEP = [10, 13, 16, 18, 20, 22, 25, 27, 29, 31][:K_H]


def _gemm_headsend(x, W):

    def body(x_ref, w_ref, e_ref, zinv_ref, nbr_raw_ref, nbr_raw_x_ref,
             s_ref, head, stat_buf, stat_recv,
             hd_send_sems, hd_recv_sems, fwdr_send_sems, fwdr_recv_sems,
             stat_send_sem, stat_recv_sem):
        j = pl.program_id(0)
        my_x = lax.axis_index("x")
        my_y = lax.axis_index("y")
        ynbr = (my_x, 1 - my_y)
        xnbr = (1 - my_x, my_y)

        e = jnp.exp(jnp.dot(x_ref[...], w_ref[...],
                            preferred_element_type=jnp.float32))
        e_ref[...] = e
        s_t = jnp.sum(e, axis=1, keepdims=True)

        @pl.when(j == 0)
        def _():
            s_ref[...] = jnp.broadcast_to(s_t, (T, SLANES))

        @pl.when(j > 0)
        def _():
            s_ref[...] = s_ref[...] + jnp.broadcast_to(s_t, (T, SLANES))

        def head_rdma(k):
            return pltpu.make_async_remote_copy(
                src_ref=head.at[k],
                dst_ref=nbr_raw_ref.at[:, pl.ds(k * TILE, TILE)],
                send_sem=hd_send_sems.at[k],
                recv_sem=hd_recv_sems.at[k],
                device_id=ynbr,
                device_id_type=pl.DeviceIdType.MESH,
            )

        def fwd_rdma(k):
            return pltpu.make_async_remote_copy(
                src_ref=nbr_raw_ref.at[:, pl.ds(k * TILE, TILE)],
                dst_ref=nbr_raw_x_ref.at[:, pl.ds(k * TILE, TILE)],
                send_sem=fwdr_send_sems.at[k],
                recv_sem=fwdr_recv_sems.at[k],
                device_id=xnbr,
                device_id_type=pl.DeviceIdType.MESH,
            )

        for k in range(K_H):
            @pl.when(j == 2 * k + my_x)
            def _(k=k):
                head[k, :, :] = e
                head_rdma(k).start()

        for k in range(K_H):
            @pl.when(j == _FWD_STEP[k])
            def _(k=k):
                head_rdma(k).wait_recv()
                fwd_rdma(k).start()

        @pl.when(j == N_TILES - 1)
        def _():
            for k in range(K_H):
                head_rdma(k).wait_send()
                fwd_rdma(k).wait_send()
                fwd_rdma(k).wait_recv()
            stat_buf[...] = s_ref[...]
            rs = pltpu.make_async_remote_copy(
                src_ref=stat_buf,
                dst_ref=stat_recv,
                send_sem=stat_send_sem,
                recv_sem=stat_recv_sem,
                device_id=ynbr,
                device_id_type=pl.DeviceIdType.MESH,
            )
            rs.start()
            rs.wait()
            z = s_ref[:, :1] + stat_recv[:, :1]
            zinv_ref[...] = jnp.broadcast_to(1.0 / z, (T, SLANES))

    return pl.pallas_call(
        body,
        grid=(N_TILES,),
        in_specs=[
            pl.BlockSpec((T, D), lambda j: (0, 0)),
            pl.BlockSpec((D, TILE), lambda j: (0, j)),
        ],
        out_specs=[
            pl.BlockSpec((T, TILE), lambda j: (0, j)),
            pl.BlockSpec((T, SLANES), lambda j: (0, 0)),
            pl.BlockSpec(memory_space=pl.ANY),
            pl.BlockSpec(memory_space=pl.ANY),
        ],
        out_shape=[
            jax.ShapeDtypeStruct((T, V_HALF), jnp.float32),
            jax.ShapeDtypeStruct((T, SLANES), jnp.float32),
            jax.ShapeDtypeStruct((T, K_H * TILE), jnp.float32),
            jax.ShapeDtypeStruct((T, K_H * TILE), jnp.float32),
        ],
        scratch_shapes=[
            pltpu.VMEM((T, SLANES), jnp.float32),
            pltpu.VMEM((K_H, T, TILE), jnp.float32),
            pltpu.VMEM((T, SLANES), jnp.float32),
            pltpu.VMEM((T, SLANES), jnp.float32),
            pltpu.SemaphoreType.DMA((K_H,)),
            pltpu.SemaphoreType.DMA((K_H,)),
            pltpu.SemaphoreType.DMA((K_H,)),
            pltpu.SemaphoreType.DMA((K_H,)),
            pltpu.SemaphoreType.DMA,
            pltpu.SemaphoreType.DMA,
        ],
        compiler_params=pltpu.CompilerParams(
            has_side_effects=True, vmem_limit_bytes=100 * 1024 * 1024),
    )(x, W)


def _normalize_exchange(e_arr, zinv, nbr_raw, nbr_raw_x):

    def body(e_ref, zinv_ref, nbr_raw_ref, nbr_raw_x_ref, out_ref,
             snd, rawy_t, rawx_t, cp_sems, snd_send_sems, d_recv_sems,
             fwd_send_sems, fwd_recv_sems, rawy_ld, rawy_st, rawx_ld,
             rawx_st):
        j = pl.program_id(0)
        kp = lax.rem(j + K_H, PAIRS)
        slot = lax.rem(j, 2)
        my_x = lax.axis_index("x")
        my_y = lax.axis_index("y")
        ynbr = (my_x, 1 - my_y)
        xnbr = (1 - my_x, my_y)
        my_col = my_y * V_HALF + kp * 2 * TILE
        dcol = my_col + my_x * TILE

        def pair_cp(sl, col):
            return pltpu.make_async_copy(
                snd.at[sl], out_ref.at[:, pl.ds(col, 2 * TILE)],
                cp_sems.at[sl])

        def direct_send(sl, col, pair_idx):
            return pltpu.make_async_remote_copy(
                src_ref=snd.at[sl, :, pl.ds(my_x * TILE, TILE)],
                dst_ref=out_ref.at[:, pl.ds(col, TILE)],
                send_sem=snd_send_sems.at[sl],
                recv_sem=d_recv_sems.at[pair_idx],
                device_id=ynbr,
                device_id_type=pl.DeviceIdType.MESH,
            )

        def fwd_rdma(pair_idx):
            fcol = (1 - my_y) * V_HALF + pair_idx * 2 * TILE + my_x * TILE
            return pltpu.make_async_remote_copy(
                src_ref=out_ref.at[:, pl.ds(fcol, TILE)],
                dst_ref=out_ref.at[:, pl.ds(fcol, TILE)],
                send_sem=fwd_send_sems.at[pair_idx],
                recv_sem=fwd_recv_sems.at[pair_idx],
                device_id=xnbr,
                device_id_type=pl.DeviceIdType.MESH,
            )

        @pl.when(j >= 2)
        def _():
            kp2 = lax.rem(j - 2 + K_H, PAIRS)
            col2 = my_y * V_HALF + kp2 * 2 * TILE
            pair_cp(slot, col2).wait()

            @pl.when(j - 2 < D_N)
            def _():
                direct_send(slot, col2 + my_x * TILE, kp2).wait_send()

        snd[slot, :, :] = e_ref[...] * zinv_ref[:, :1]
        pair_cp(slot, my_col).start()

        @pl.when(j < D_N)
        def _():
            direct_send(slot, dcol, kp).start()

        @pl.when((j >= 2) & (j < 2 + D_N))
        def _():
            kf = j - 2 + K_H
            fcol = (1 - my_y) * V_HALF + kf * 2 * TILE + my_x * TILE
            arr = pltpu.make_async_remote_copy(
                src_ref=snd.at[0, :, pl.ds(0, TILE)],
                dst_ref=out_ref.at[:, pl.ds(fcol, TILE)],
                send_sem=snd_send_sems.at[0],
                recv_sem=d_recv_sems.at[kf],
                device_id=ynbr,
                device_id_type=pl.DeviceIdType.MESH,
            )
            arr.wait_recv()
            fwd_rdma(kf).start()

        @pl.when(j >= D_N)
        def _():
            ycol = (1 - my_y) * V_HALF + (2 * kp + my_x) * TILE
            ld = pltpu.make_async_copy(
                nbr_raw_ref.at[:, pl.ds(kp * TILE, TILE)], rawy_t, rawy_ld)
            ld.start()
            ld.wait()
            rawy_t[...] = rawy_t[...] * zinv_ref[:, :1]
            st = pltpu.make_async_copy(
                rawy_t, out_ref.at[:, pl.ds(ycol, TILE)], rawy_st)
            st.start()
            xcol = (1 - my_y) * V_HALF + (2 * kp + 1 - my_x) * TILE
            ld2 = pltpu.make_async_copy(
                nbr_raw_x_ref.at[:, pl.ds(kp * TILE, TILE)], rawx_t, rawx_ld)
            ld2.start()
            ld2.wait()
            rawx_t[...] = rawx_t[...] * zinv_ref[:, :1]
            st2 = pltpu.make_async_copy(
                rawx_t, out_ref.at[:, pl.ds(xcol, TILE)], rawx_st)
            st2.start()
            st.wait()
            st2.wait()

        @pl.when(j == PAIRS - 1)
        def _():
            for dj in (PAIRS - 2, PAIRS - 1):
                sl = dj % 2
                kpd = (dj + K_H) % PAIRS
                pair_cp(sl, my_y * V_HALF + kpd * 2 * TILE).wait()
            for k in range(K_H, PAIRS):
                fwd_rdma(k).wait_send()
                fcol_in = ((1 - my_y) * V_HALF + k * 2 * TILE
                           + (1 - my_x) * TILE)
                arr = pltpu.make_async_remote_copy(
                    src_ref=snd.at[0, :, pl.ds(0, TILE)],
                    dst_ref=out_ref.at[:, pl.ds(fcol_in, TILE)],
                    send_sem=snd_send_sems.at[0],
                    recv_sem=fwd_recv_sems.at[k],
                    device_id=xnbr,
                    device_id_type=pl.DeviceIdType.MESH,
                )
                arr.wait_recv()

    return pl.pallas_call(
        body,
        grid=(PAIRS,),
        in_specs=[
            pl.BlockSpec((T, 2 * TILE), lambda j: (0, (j + K_H) % PAIRS)),
            pl.BlockSpec((T, SLANES), lambda j: (0, 0)),
            pl.BlockSpec(memory_space=pl.ANY),
            pl.BlockSpec(memory_space=pl.ANY),
        ],
        out_specs=pl.BlockSpec(memory_space=pl.ANY),
        out_shape=jax.ShapeDtypeStruct((T, V), jnp.float32),
        scratch_shapes=[
            pltpu.VMEM((2, T, 2 * TILE), jnp.float32),
            pltpu.VMEM((T, TILE), jnp.float32),
            pltpu.VMEM((T, TILE), jnp.float32),
            pltpu.SemaphoreType.DMA((2,)),
            pltpu.SemaphoreType.DMA((2,)),
            pltpu.SemaphoreType.DMA((PAIRS,)),
            pltpu.SemaphoreType.DMA((PAIRS,)),
            pltpu.SemaphoreType.DMA((PAIRS,)),
            pltpu.SemaphoreType.DMA,
            pltpu.SemaphoreType.DMA,
            pltpu.SemaphoreType.DMA,
            pltpu.SemaphoreType.DMA,
        ],
        compiler_params=pltpu.CompilerParams(
            has_side_effects=True, vmem_limit_bytes=64 * 1024 * 1024),
    )(e_arr, zinv, nbr_raw, nbr_raw_x)


def kernel(x, W):
    e_arr, zinv, nbr_raw, nbr_raw_x = _gemm_headsend(x, W)
    return _normalize_exchange(e_arr, zinv, nbr_raw, nbr_raw_x)
